# baseline (speedup 1.0000x reference)
import functools

import jax
import jax.numpy as jnp
from jax import lax
from jax.experimental import pallas as pl
from jax.experimental.pallas import tpu as pltpu
from jax.experimental.pallas import tpu_sc as plsc

B, C, U, F = 4, 24, 16, 4
E, H, DS, DE = 128, 128, 64, 64
VW, VS, VE = 100000, 10, 8
SH = E // 2
BC = B * C
NARC = B * C * C
NIDX = B * C * U * F


def _sc_gather(table, idx):
    info = plsc.get_sparse_core_info()
    nc, ns = info.num_cores, info.num_subcores
    nw = nc * ns
    bpw = NIDX // nw
    mesh = plsc.VectorSubcoreMesh(core_axis_name="c", subcore_axis_name="s")

    @functools.partial(
        pl.kernel,
        mesh=mesh,
        out_type=jax.ShapeDtypeStruct((NIDX, E), jnp.float32),
        scratch_types=[
            pltpu.VMEM((bpw,), jnp.int32),
            pltpu.VMEM((bpw, E), jnp.float32),
            pltpu.SemaphoreType.DMA,
        ],
    )
    def k(table_hbm, idx_hbm, out_hbm, idx_v, rows_v, sem):
        wid = lax.axis_index("s") * nc + lax.axis_index("c")
        base = wid * bpw
        pltpu.sync_copy(idx_hbm.at[pl.ds(base, bpw)], idx_v)
        pltpu.async_copy(table_hbm.at[idx_v], rows_v, sem).wait()
        pltpu.sync_copy(rows_v, out_hbm.at[pl.ds(base, bpw)])

    return k(table, idx)


def _sig(x):
    return 0.5 * jnp.tanh(0.5 * x) + 0.5


def _dot(a, b, prec=None):
    return jax.lax.dot_general(a, b, (((1,), (0,)), ((), ())),
                               precision=prec,
                               preferred_element_type=jnp.float32)


def _dot_t(a, b, prec=None):
    return jax.lax.dot_general(a, b, (((1,), (1,)), ((), ())),
                               precision=prec,
                               preferred_element_type=jnp.float32)


_HI = jax.lax.Precision.HIGHEST


_GATE_ORDER = (0, 1, 3, 2)


def _pack_rec(mf, mb, w, z):
    parts = []
    for gidx in _GATE_ORDER:
        lo = gidx * w
        parts.append(jnp.concatenate([mf[lo:lo + w], z], axis=1))
        parts.append(jnp.concatenate([z, mb[lo:lo + w]], axis=1))
    return jnp.concatenate(parts, axis=0)


def _pack_rows(mf, mb, w):
    parts = []
    for gidx in _GATE_ORDER:
        lo = gidx * w
        parts.append(mf[lo:lo + w])
        parts.append(mb[lo:lo + w])
    return jnp.concatenate(parts, axis=0)


def _pack_half(m, w, z, fwd_live):
    parts = []
    for gidx in _GATE_ORDER:
        lo = gidx * w
        if fwd_live:
            parts.append(m[lo:lo + w])
            parts.append(z)
        else:
            parts.append(z)
            parts.append(m[lo:lo + w])
    return jnp.concatenate(parts, axis=0)


def _fused_body(g_ref, spk_ids_ref, em_ids_ref, gcol_ref,
                utWihf_ref, utWihb_ref, utWhhf_ref, utWhhb_ref,
                utbf_ref, utbb_ref,
                spk_tab_ref, em_tab_ref,
                wc_ref, wcb_ref, we_ref, web_ref,
                emc_ref, emcb_ref, eme_ref, emeb_ref,
                wut_ref, wem_ref,
                spWihf_ref, spWihb_ref, spWhhf_ref, spWhhb_ref,
                spbf_ref, spbb_ref, spow_ref, spob_ref,
                sut_ref, sem_ref, sp_ref):
    g = g_ref[...]
    we_all = (g[0] + g[1] + g[2] + g[3]) * 0.25
    we3 = we_all.reshape(U, BC, E)
    xs = [we3[t] for t in range(U)]

    def ut_lstm(wih_ref, whh_ref, b_ref, reverse):
        wih = wih_ref[...]
        whh = whh_ref[...]
        bb = b_ref[...]
        h = jnp.zeros((BC, H), jnp.float32)
        cst = jnp.zeros((BC, H), jnp.float32)
        order = range(U - 1, -1, -1) if reverse else range(U)
        for t in order:
            gates = _dot_t(xs[t], wih) + _dot_t(h, whh) + bb
            gi = jax.nn.sigmoid(gates[:, 0:H])
            gf = jax.nn.sigmoid(gates[:, H:2 * H])
            gg = jnp.tanh(gates[:, 2 * H:3 * H])
            go = jax.nn.sigmoid(gates[:, 3 * H:4 * H])
            cst = gf * cst + gi * gg
            h = go * jnp.tanh(cst)
        return h

    hT_f = ut_lstm(utWihf_ref, utWhhf_ref, utbf_ref, False)
    hT_b = ut_lstm(utWihb_ref, utWhhb_ref, utbb_ref, True)
    h2 = jnp.concatenate([hT_f, hT_b], axis=1)

    spk_oh = (spk_ids_ref[...] ==
              jax.lax.broadcasted_iota(jnp.int32, (BC, VS), 1)
              ).astype(jnp.float32)
    spk = _dot(spk_oh, spk_tab_ref[...], _HI)
    em_oh = (em_ids_ref[...] ==
             jax.lax.broadcasted_iota(jnp.int32, (BC, VE), 1)
             ).astype(jnp.float32)
    em_e = _dot(em_oh, em_tab_ref[...], _HI)

    ut = jnp.concatenate([h2, spk], axis=-1)

    def ffn(wref, bref):
        y = _dot_t(ut, wref[...]) + bref[...]
        return jnp.where(y >= 0, y, 0.1 * y)

    ut_cause = ffn(wc_ref, wcb_ref)
    ut_effect = ffn(we_ref, web_ref)
    em_cause = ffn(emc_ref, emcb_ref)
    em_effect = ffn(eme_ref, emeb_ref)

    ones = jnp.ones((C, 1), jnp.float32)
    wut = wut_ref[0]
    wem = wem_ref[...]
    t1_rows = []
    for bb in range(B):
        r0 = bb * C
        xe = jnp.concatenate([ut_effect[r0:r0 + C], ones], axis=-1)
        yc = ut_cause[r0:r0 + C]
        t1 = _dot(xe, wut)
        t1_rows.append(t1)
        sut_ref[bb] = _dot_t(t1, yc)
        xem = jnp.concatenate([em_effect[r0:r0 + C], ones], axis=-1)
        yem = jnp.concatenate([em_cause[r0:r0 + C], ones], axis=-1)
        for o in range(VE):
            t2 = _dot(xem, wem[o])
            sem_ref[bb, o] = _dot_t(t2, yem)

    def expand_cau(x):
        w = x.shape[1]
        x4 = x.reshape(B, 1, C, w)
        return jnp.broadcast_to(x4, (B, C, C, w)).reshape(NARC, w)

    def expand_eff(x):
        w = x.shape[1]
        x4 = x.reshape(B, C, 1, w)
        return jnp.broadcast_to(x4, (B, C, C, w)).reshape(NARC, w)

    spWihf = spWihf_ref[...]
    spWihb = spWihb_ref[...]
    zS = jnp.zeros((SH, E), jnp.float32)
    spAf = _pack_half(spWihf[:, 0:E], SH, zS, True)
    spAb = _pack_half(spWihb[:, 0:E], SH, zS, False)
    spE2 = _pack_rows(spWihf[:, E:], spWihb[:, E:], SH)

    def pack_cols_sp(a, b):
        return jnp.concatenate([
            a[:, 0:SH], b[:, 0:SH],
            a[:, SH:2 * SH], b[:, SH:2 * SH],
            a[:, 3 * SH:4 * SH], b[:, 3 * SH:4 * SH],
            a[:, 2 * SH:3 * SH], b[:, 2 * SH:3 * SH],
        ], axis=1)

    spb2 = pack_cols_sp(spbf_ref[...], spbb_ref[...])
    zSh = jnp.zeros((SH, SH), jnp.float32)
    spR2 = _pack_rec(spWhhf_ref[...], spWhhb_ref[...], SH, zSh)

    xw2 = [_dot_t(xs[s], spAf) + _dot_t(xs[U - 1 - s], spAb)
           for s in range(U)]
    eb2 = expand_eff(_dot_t(em_e, spE2) + spb2)

    spow = spow_ref[...]
    z1 = jnp.zeros((1, SH), jnp.float32)
    w2 = jnp.concatenate([
        jnp.concatenate([spow[:, 0:SH], z1], axis=1),
        jnp.concatenate([z1, spow[:, SH:2 * SH]], axis=1),
    ], axis=0)

    h2s = jnp.zeros((NARC, 2 * SH), jnp.float32)
    c2s = jnp.zeros((NARC, 2 * SH), jnp.float32)
    fcol = [None] * U
    bcol = [None] * U
    for s in range(U):
        gates = expand_cau(xw2[s]) + eb2 + _dot_t(h2s, spR2)
        sg = _sig(gates[:, 0:384])
        gg = jnp.tanh(gates[:, 384:512])
        c2s = sg[:, 128:256] * c2s + sg[:, 0:128] * gg
        h2s = sg[:, 256:384] * jnp.tanh(c2s)
        p2 = _dot_t(h2s, w2)
        fcol[s] = p2[:, 0:1]
        bcol[U - 1 - s] = p2[:, 1:2]

    logit = jnp.concatenate([fcol[t] + bcol[t] for t in range(U)],
                            axis=1) + spob_ref[...]
    preds = _sig(logit)

    t196 = jnp.concatenate(t1_rows, axis=0)
    ta = expand_eff(t196).astype(jnp.bfloat16).astype(jnp.float32)
    tb = expand_cau(ut_cause).astype(jnp.bfloat16).astype(jnp.float32)
    s_col = jnp.sum(ta * tb, axis=1, keepdims=True)
    mask = (gcol_ref[...] != 0) | (s_col > 0.0)
    sp_ref[...] = preds * mask.astype(jnp.float32)


def _fused_call(g, spk_ids, em_ids, gcol, p):
    out_shapes = [
        jax.ShapeDtypeStruct((B, C, C), jnp.float32),
        jax.ShapeDtypeStruct((B, VE, C, C), jnp.float32),
        jax.ShapeDtypeStruct((NARC, U), jnp.float32),
    ]
    args = [
        g.reshape(F, U * BC, E),
        spk_ids, em_ids, gcol,
        p['ut_Wih_f'], p['ut_Wih_b'], p['ut_Whh_f'], p['ut_Whh_b'],
        p['ut_b_f'].reshape(1, -1), p['ut_b_b'].reshape(1, -1),
        p['spk_table'], p['em_table'],
        p['ut_cause_W'], p['ut_cause_b'].reshape(1, -1),
        p['ut_effect_W'], p['ut_effect_b'].reshape(1, -1),
        p['em_cause_W'], p['em_cause_b'].reshape(1, -1),
        p['em_effect_W'], p['em_effect_b'].reshape(1, -1),
        p['W_ut'], p['W_em'],
        p['sp_Wih_f'], p['sp_Wih_b'], p['sp_Whh_f'], p['sp_Whh_b'],
        p['sp_b_f'].reshape(1, -1), p['sp_b_b'].reshape(1, -1),
        p['sp_out_W'], p['sp_out_b'].reshape(1, 1),
    ]
    return pl.pallas_call(_fused_body, out_shape=out_shapes)(*args)


def kernel(words, speakers, emotions, graphs, spans, params):
    del spans
    idx = words.astype(jnp.int32).transpose(3, 2, 0, 1).reshape(-1)
    g = _sc_gather(params['word_table'], idx)

    spk_ids = speakers.astype(jnp.int32).reshape(BC, 1)
    em_ids = emotions.astype(jnp.int32).reshape(BC, 1)
    gcol = graphs.astype(jnp.int32).reshape(NARC, 1)
    s_ut, s_em_k, sp = _fused_call(g, spk_ids, em_ids, gcol, params)

    s_em = jnp.transpose(s_em_k, (0, 2, 3, 1))
    s_span = sp.reshape(B, C, C, U)
    return (s_ut, s_em, s_span)


# speedup vs baseline: 1.0068x; 1.0068x over previous
import functools

import jax
import jax.numpy as jnp
from jax import lax
from jax.experimental import pallas as pl
from jax.experimental.pallas import tpu as pltpu
from jax.experimental.pallas import tpu_sc as plsc

B, C, U, F = 4, 24, 16, 4
E, H, DS, DE = 128, 128, 64, 64
VW, VS, VE = 100000, 10, 8
SH = E // 2
BC = B * C
NARC = B * C * C
NIDX = B * C * U * F


def _sc_gather_mean(table, idx):
    info = plsc.get_sparse_core_info()
    nc, ns = info.num_cores, info.num_subcores
    nw = nc * ns
    bpw = NIDX // nw
    opw = bpw // F
    nlanes = E // 16
    mesh = plsc.VectorSubcoreMesh(core_axis_name="c", subcore_axis_name="s")

    @functools.partial(
        pl.kernel,
        mesh=mesh,
        out_type=jax.ShapeDtypeStruct((NIDX // F, E), jnp.float32),
        scratch_types=[
            pltpu.VMEM((bpw,), jnp.int32),
            pltpu.VMEM((bpw, E), jnp.float32),
            pltpu.VMEM((opw, E), jnp.float32),
            pltpu.SemaphoreType.DMA,
        ],
    )
    def k(table_hbm, idx_hbm, out_hbm, idx_v, rows_v, mean_v, sem):
        wid = lax.axis_index("s") * nc + lax.axis_index("c")
        pltpu.sync_copy(idx_hbm.at[pl.ds(wid * bpw, bpw)], idx_v)
        pltpu.async_copy(table_hbm.at[idx_v], rows_v, sem).wait()

        def row_body(q, carry):
            for l in range(nlanes):
                sl = pl.ds(16 * l, 16)
                acc = (rows_v[4 * q, sl] + rows_v[4 * q + 1, sl]
                       + rows_v[4 * q + 2, sl] + rows_v[4 * q + 3, sl])
                mean_v[q, sl] = acc * 0.25
            return carry

        lax.fori_loop(0, opw, row_body, 0)
        pltpu.sync_copy(mean_v, out_hbm.at[pl.ds(wid * opw, opw)])

    return k(table, idx)


def _sig(x):
    return 0.5 * jnp.tanh(0.5 * x) + 0.5


def _dot(a, b, prec=None):
    return jax.lax.dot_general(a, b, (((1,), (0,)), ((), ())),
                               precision=prec,
                               preferred_element_type=jnp.float32)


def _dot_t(a, b, prec=None):
    return jax.lax.dot_general(a, b, (((1,), (1,)), ((), ())),
                               precision=prec,
                               preferred_element_type=jnp.float32)


_HI = jax.lax.Precision.HIGHEST


_GATE_ORDER = (0, 1, 3, 2)


def _pack_rec(mf, mb, w, z):
    parts = []
    for gidx in _GATE_ORDER:
        lo = gidx * w
        parts.append(jnp.concatenate([mf[lo:lo + w], z], axis=1))
        parts.append(jnp.concatenate([z, mb[lo:lo + w]], axis=1))
    return jnp.concatenate(parts, axis=0)


def _pack_rows(mf, mb, w):
    parts = []
    for gidx in _GATE_ORDER:
        lo = gidx * w
        parts.append(mf[lo:lo + w])
        parts.append(mb[lo:lo + w])
    return jnp.concatenate(parts, axis=0)


def _pack_half(m, w, z, fwd_live):
    parts = []
    for gidx in _GATE_ORDER:
        lo = gidx * w
        if fwd_live:
            parts.append(m[lo:lo + w])
            parts.append(z)
        else:
            parts.append(z)
            parts.append(m[lo:lo + w])
    return jnp.concatenate(parts, axis=0)


def _fused_body(g_ref, spk_ids_ref, em_ids_ref, gcol_ref,
                utWihf_ref, utWihb_ref, utWhhf_ref, utWhhb_ref,
                utbf_ref, utbb_ref,
                spk_tab_ref, em_tab_ref,
                wc_ref, wcb_ref, we_ref, web_ref,
                emc_ref, emcb_ref, eme_ref, emeb_ref,
                wut_ref, wem_ref,
                spWihf_ref, spWihb_ref, spWhhf_ref, spWhhb_ref,
                spbf_ref, spbb_ref, spow_ref, spob_ref,
                sut_ref, sem_ref, sp_ref):
    we3 = g_ref[...].reshape(U, BC, E)
    xs = [we3[t] for t in range(U)]

    def ut_lstm(wih_ref, whh_ref, b_ref, reverse):
        wih = wih_ref[...]
        whh = whh_ref[...]
        bb = b_ref[...]
        h = jnp.zeros((BC, H), jnp.float32)
        cst = jnp.zeros((BC, H), jnp.float32)
        order = range(U - 1, -1, -1) if reverse else range(U)
        for t in order:
            gates = _dot_t(xs[t], wih) + _dot_t(h, whh) + bb
            gi = jax.nn.sigmoid(gates[:, 0:H])
            gf = jax.nn.sigmoid(gates[:, H:2 * H])
            gg = jnp.tanh(gates[:, 2 * H:3 * H])
            go = jax.nn.sigmoid(gates[:, 3 * H:4 * H])
            cst = gf * cst + gi * gg
            h = go * jnp.tanh(cst)
        return h

    hT_f = ut_lstm(utWihf_ref, utWhhf_ref, utbf_ref, False)
    hT_b = ut_lstm(utWihb_ref, utWhhb_ref, utbb_ref, True)
    h2 = jnp.concatenate([hT_f, hT_b], axis=1)

    spk_oh = (spk_ids_ref[...] ==
              jax.lax.broadcasted_iota(jnp.int32, (BC, VS), 1)
              ).astype(jnp.float32)
    spk = _dot(spk_oh, spk_tab_ref[...], _HI)
    em_oh = (em_ids_ref[...] ==
             jax.lax.broadcasted_iota(jnp.int32, (BC, VE), 1)
             ).astype(jnp.float32)
    em_e = _dot(em_oh, em_tab_ref[...], _HI)

    ut = jnp.concatenate([h2, spk], axis=-1)

    def ffn(wref, bref):
        y = _dot_t(ut, wref[...]) + bref[...]
        return jnp.where(y >= 0, y, 0.1 * y)

    ut_cause = ffn(wc_ref, wcb_ref)
    ut_effect = ffn(we_ref, web_ref)
    em_cause = ffn(emc_ref, emcb_ref)
    em_effect = ffn(eme_ref, emeb_ref)

    ones = jnp.ones((C, 1), jnp.float32)
    wut = wut_ref[0]
    wem = wem_ref[...]
    t1_rows = []
    for bb in range(B):
        r0 = bb * C
        xe = jnp.concatenate([ut_effect[r0:r0 + C], ones], axis=-1)
        yc = ut_cause[r0:r0 + C]
        t1 = _dot(xe, wut)
        t1_rows.append(t1)
        sut_ref[bb] = _dot_t(t1, yc)
        xem = jnp.concatenate([em_effect[r0:r0 + C], ones], axis=-1)
        yem = jnp.concatenate([em_cause[r0:r0 + C], ones], axis=-1)
        for o in range(VE):
            t2 = _dot(xem, wem[o])
            sem_ref[bb, o] = _dot_t(t2, yem)

    def expand_cau(x):
        w = x.shape[1]
        x4 = x.reshape(B, 1, C, w)
        return jnp.broadcast_to(x4, (B, C, C, w)).reshape(NARC, w)

    def expand_eff(x):
        w = x.shape[1]
        x4 = x.reshape(B, C, 1, w)
        return jnp.broadcast_to(x4, (B, C, C, w)).reshape(NARC, w)

    spWihf = spWihf_ref[...]
    spWihb = spWihb_ref[...]
    zS = jnp.zeros((SH, E), jnp.float32)
    spAf = _pack_half(spWihf[:, 0:E], SH, zS, True)
    spAb = _pack_half(spWihb[:, 0:E], SH, zS, False)
    spE2 = _pack_rows(spWihf[:, E:], spWihb[:, E:], SH)

    def pack_cols_sp(a, b):
        return jnp.concatenate([
            a[:, 0:SH], b[:, 0:SH],
            a[:, SH:2 * SH], b[:, SH:2 * SH],
            a[:, 3 * SH:4 * SH], b[:, 3 * SH:4 * SH],
            a[:, 2 * SH:3 * SH], b[:, 2 * SH:3 * SH],
        ], axis=1)

    spb2 = pack_cols_sp(spbf_ref[...], spbb_ref[...])
    zSh = jnp.zeros((SH, SH), jnp.float32)
    spR2 = _pack_rec(spWhhf_ref[...], spWhhb_ref[...], SH, zSh)

    xw2 = [_dot_t(xs[s], spAf) + _dot_t(xs[U - 1 - s], spAb)
           for s in range(U)]
    eb2 = expand_eff(_dot_t(em_e, spE2) + spb2)

    spow = spow_ref[...]
    z1 = jnp.zeros((1, SH), jnp.float32)
    w2 = jnp.concatenate([
        jnp.concatenate([spow[:, 0:SH], z1], axis=1),
        jnp.concatenate([z1, spow[:, SH:2 * SH]], axis=1),
    ], axis=0)

    h2s = jnp.zeros((NARC, 2 * SH), jnp.float32)
    c2s = jnp.zeros((NARC, 2 * SH), jnp.float32)
    fcol = [None] * U
    bcol = [None] * U
    for s in range(U):
        gates = expand_cau(xw2[s]) + eb2 + _dot_t(h2s, spR2)
        sg = _sig(gates[:, 0:384])
        gg = jnp.tanh(gates[:, 384:512])
        c2s = sg[:, 128:256] * c2s + sg[:, 0:128] * gg
        h2s = sg[:, 256:384] * jnp.tanh(c2s)
        p2 = _dot_t(h2s, w2)
        fcol[s] = p2[:, 0:1]
        bcol[U - 1 - s] = p2[:, 1:2]

    logit = jnp.concatenate([fcol[t] + bcol[t] for t in range(U)],
                            axis=1) + spob_ref[...]
    preds = _sig(logit)

    t196 = jnp.concatenate(t1_rows, axis=0)
    ta = expand_eff(t196).astype(jnp.bfloat16).astype(jnp.float32)
    tb = expand_cau(ut_cause).astype(jnp.bfloat16).astype(jnp.float32)
    s_col = jnp.sum(ta * tb, axis=1, keepdims=True)
    mask = (gcol_ref[...] != 0) | (s_col > 0.0)
    sp_ref[...] = preds * mask.astype(jnp.float32)


def _fused_call(g, spk_ids, em_ids, gcol, p):
    out_shapes = [
        jax.ShapeDtypeStruct((B, C, C), jnp.float32),
        jax.ShapeDtypeStruct((B, VE, C, C), jnp.float32),
        jax.ShapeDtypeStruct((NARC, U), jnp.float32),
    ]
    args = [
        g,
        spk_ids, em_ids, gcol,
        p['ut_Wih_f'], p['ut_Wih_b'], p['ut_Whh_f'], p['ut_Whh_b'],
        p['ut_b_f'].reshape(1, -1), p['ut_b_b'].reshape(1, -1),
        p['spk_table'], p['em_table'],
        p['ut_cause_W'], p['ut_cause_b'].reshape(1, -1),
        p['ut_effect_W'], p['ut_effect_b'].reshape(1, -1),
        p['em_cause_W'], p['em_cause_b'].reshape(1, -1),
        p['em_effect_W'], p['em_effect_b'].reshape(1, -1),
        p['W_ut'], p['W_em'],
        p['sp_Wih_f'], p['sp_Wih_b'], p['sp_Whh_f'], p['sp_Whh_b'],
        p['sp_b_f'].reshape(1, -1), p['sp_b_b'].reshape(1, -1),
        p['sp_out_W'], p['sp_out_b'].reshape(1, 1),
    ]
    return pl.pallas_call(_fused_body, out_shape=out_shapes)(*args)


def kernel(words, speakers, emotions, graphs, spans, params):
    del spans
    idx = words.astype(jnp.int32).transpose(2, 0, 1, 3).reshape(-1)
    g = _sc_gather_mean(params['word_table'], idx)

    spk_ids = speakers.astype(jnp.int32).reshape(BC, 1)
    em_ids = emotions.astype(jnp.int32).reshape(BC, 1)
    gcol = graphs.astype(jnp.int32).reshape(NARC, 1)
    s_ut, s_em_k, sp = _fused_call(g, spk_ids, em_ids, gcol, params)

    s_em = jnp.transpose(s_em_k, (0, 2, 3, 1))
    s_span = sp.reshape(B, C, C, U)
    return (s_ut, s_em, s_span)
